# baseline (speedup 1.0000x reference)
import jax
import jax.numpy as jnp
from jax.experimental import pallas as pl
from jax.experimental.pallas import tpu as pltpu

ROWS = 4096
DIM = 64
QCOL = 1024
NQ = ROWS // QCOL
_ALPHA = 0.1
_BIG = 1e30


def _enc_labels(lab):
    v = (128 + (lab & 127)) << (lab >> 7)
    return v.astype(jnp.float32).astype(jnp.bfloat16)


def _triplet_kernel(h_ref, lab_row_ref, lab_col_ref, out_ref):
    h32 = h_ref[...]
    h = h32.astype(jnp.bfloat16)
    hneg2 = h * jnp.bfloat16(-2.0)

    lab_row = _enc_labels(lab_row_ref[...])
    lab_col = _enc_labels(lab_col_ref[...])

    ones = jnp.ones((1, DIM), dtype=jnp.bfloat16)
    xn_cols = jax.lax.dot_general(
        ones, h * h, (((1,), (1,)), ((), ())),
        preferred_element_type=jnp.float32).astype(jnp.bfloat16)

    big = jnp.bfloat16(_BIG)
    posv = None
    m1 = None
    m2 = None
    e0 = None
    for q in range(NQ):
        rs = slice(q * QCOL, (q + 1) * QCOL)
        s2q = jax.lax.dot_general(
            hneg2, h[rs], (((1,), (1,)), ((), ())),
            preferred_element_type=jnp.float32)
        eqv = xn_cols[:, rs] + s2q.astype(jnp.bfloat16)
        if q == 0:
            e0 = eqv[:, 0:1]
        eqm = lab_row == lab_col[:, rs]
        pv = jnp.max(jnp.where(eqm, eqv, -big), axis=1, keepdims=True)
        nv = jnp.where(eqm, big, eqv)
        m1q = jnp.min(nv, axis=1, keepdims=True)
        m2q = jnp.min(jnp.where(nv > m1q, nv, big), axis=1, keepdims=True)
        if q == 0:
            posv, m1, m2 = pv, m1q, m2q
        else:
            posv = jnp.maximum(posv, pv)
            m2 = jnp.minimum(jnp.minimum(m2, m2q), jnp.maximum(m1, m1q))
            m1 = jnp.minimum(m1, m1q)

    no_pos_thresh = (2.0 - jnp.sum(jnp.square(h32),
                                   axis=1, keepdims=True)).astype(jnp.bfloat16)
    p = jnp.where(posv <= no_pos_thresh, e0, posv)

    t = jnp.maximum(p.astype(jnp.float32) - m2.astype(jnp.float32) + _ALPHA,
                    0.0)
    live = t > 1e-7
    bs = jnp.sum(jnp.where(live, t, 0.0))
    bc = jnp.sum(live.astype(jnp.float32))
    out_ref[0] = bs / bc


def kernel(H, labels):
    lab_row = labels.reshape(ROWS, 1)
    lab_col = labels.reshape(1, ROWS)
    out = pl.pallas_call(
        _triplet_kernel,
        grid=(1,),
        in_specs=[
            pl.BlockSpec((ROWS, DIM), lambda i: (0, 0)),
            pl.BlockSpec((ROWS, 1), lambda i: (0, 0)),
            pl.BlockSpec((1, ROWS), lambda i: (0, 0)),
        ],
        out_specs=pl.BlockSpec(memory_space=pltpu.SMEM),
        out_shape=jax.ShapeDtypeStruct((1,), jnp.float32),
    )(H, lab_row, lab_col)
    return out[0]


# speedup vs baseline: 1.0403x; 1.0403x over previous
import jax
import jax.numpy as jnp
from jax.experimental import pallas as pl
from jax.experimental.pallas import tpu as pltpu

ROWS = 4096
DIM = 64
BLOCK = 2048
NBLK = ROWS // BLOCK
_ALPHA = 0.1
_BIG = 1e30


def _enc_labels(lab):
    v = (128 + (lab & 127)) << (lab >> 7)
    return v.astype(jnp.float32).astype(jnp.bfloat16)


def _triplet_kernel(h_rows_ref, h_full_ref, lab_row_ref, lab_col_ref,
                    out_ref, acc_ref):
    i = pl.program_id(0)

    h_rows32 = h_rows_ref[...]
    h_rows = h_rows32.astype(jnp.bfloat16)
    h_full = h_full_ref[...].astype(jnp.bfloat16)

    ones = jnp.ones((1, DIM), dtype=jnp.bfloat16)
    xn_cols = jax.lax.dot_general(
        ones, h_full * h_full, (((1,), (1,)), ((), ())),
        preferred_element_type=jnp.float32)
    s2 = jax.lax.dot_general(
        h_rows * jnp.bfloat16(-2.0), h_full, (((1,), (1,)), ((), ())),
        preferred_element_type=jnp.float32)
    e = xn_cols.astype(jnp.bfloat16) + s2.astype(jnp.bfloat16)

    lab_row = _enc_labels(lab_row_ref[...])
    lab_col = _enc_labels(lab_col_ref[...])
    eq = lab_row == lab_col

    big = jnp.bfloat16(_BIG)
    posv = jnp.max(jnp.where(eq, e, -big), axis=1, keepdims=True)
    no_pos_thresh = (2.0 - jnp.sum(jnp.square(h_rows32),
                                   axis=1, keepdims=True)).astype(jnp.bfloat16)
    p = jnp.where(posv <= no_pos_thresh, e[:, 0:1], posv)

    negv = jnp.where(eq, big, e)
    m1 = jnp.min(negv, axis=1, keepdims=True)
    m2 = jnp.min(jnp.where(negv > m1, negv, big), axis=1, keepdims=True)

    t = jnp.maximum(p.astype(jnp.float32) - m2.astype(jnp.float32) + _ALPHA,
                    0.0)
    live = t > 1e-7
    bs = jnp.sum(jnp.where(live, t, 0.0))
    bc = jnp.sum(live.astype(jnp.float32))

    @pl.when(i == 0)
    def _init():
        acc_ref[0, 0] = 0.0
        acc_ref[0, 1] = 0.0

    acc_ref[0, 0] += bs
    acc_ref[0, 1] += bc

    @pl.when(i == NBLK - 1)
    def _fin():
        out_ref[...] = jnp.full((1, 1), acc_ref[0, 0] / acc_ref[0, 1],
                                dtype=jnp.float32)


def kernel(H, labels):
    lab_row = labels.reshape(ROWS, 1)
    lab_col = labels.reshape(1, ROWS)
    out = pl.pallas_call(
        _triplet_kernel,
        grid=(NBLK,),
        in_specs=[
            pl.BlockSpec((BLOCK, DIM), lambda i: (i, 0)),
            pl.BlockSpec((ROWS, DIM), lambda i: (0, 0)),
            pl.BlockSpec((BLOCK, 1), lambda i: (i, 0)),
            pl.BlockSpec((1, ROWS), lambda i: (0, 0)),
        ],
        out_specs=pl.BlockSpec((1, 1), lambda i: (0, 0)),
        out_shape=jax.ShapeDtypeStruct((1, 1), jnp.float32),
        scratch_shapes=[pltpu.SMEM((1, 2), jnp.float32)],
    )(H, H, lab_row, lab_col)
    return out[0, 0]


# speedup vs baseline: 1.0430x; 1.0026x over previous
import jax
import jax.numpy as jnp
from jax.experimental import pallas as pl
from jax.experimental.pallas import tpu as pltpu

ROWS = 4096
DIM = 64
BLOCK = 2048
NBLK = ROWS // BLOCK
_ALPHA = 0.1
_BIG = 1e30


def _enc_labels(lab):
    v = (128 + (lab & 127)) << (lab >> 7)
    return v.astype(jnp.float32).astype(jnp.bfloat16)


def _triplet_kernel(h_rows_ref, h_full_ref, lab_row_ref, lab_col_ref,
                    out_ref, acc_ref):
    i = pl.program_id(0)

    h_rows32 = h_rows_ref[...]
    h_rows = h_rows32.astype(jnp.bfloat16)
    h_full32 = h_full_ref[...]
    h_full = h_full32.astype(jnp.bfloat16)

    xn_col = jnp.sum(jnp.square(h_full32), axis=1,
                     keepdims=True).astype(jnp.bfloat16)
    lhs = jnp.concatenate(
        [h_rows * jnp.bfloat16(-2.0),
         jnp.ones((BLOCK, 1), dtype=jnp.bfloat16)], axis=1)
    rhs = jnp.concatenate([h_full, xn_col], axis=1)
    s2 = jax.lax.dot_general(
        lhs, rhs, (((1,), (1,)), ((), ())),
        preferred_element_type=jnp.float32)
    e = s2.astype(jnp.bfloat16)

    lab_row = _enc_labels(lab_row_ref[...])
    lab_col = _enc_labels(lab_col_ref[...])
    eq = lab_row == lab_col

    big = jnp.bfloat16(_BIG)
    posv = jnp.max(jnp.where(eq, e, -big), axis=1, keepdims=True)
    no_pos_thresh = (2.0 - jnp.sum(jnp.square(h_rows32),
                                   axis=1, keepdims=True)).astype(jnp.bfloat16)
    p = jnp.where(posv <= no_pos_thresh, e[:, 0:1], posv)

    negv = jnp.where(eq, big, e)
    m1 = jnp.min(negv, axis=1, keepdims=True)
    m2 = jnp.min(jnp.where(negv > m1, negv, big), axis=1, keepdims=True)

    t = jnp.maximum(p.astype(jnp.float32) - m2.astype(jnp.float32) + _ALPHA,
                    0.0)
    live = t > 1e-7
    bs = jnp.sum(jnp.where(live, t, 0.0))
    bc = jnp.sum(live.astype(jnp.float32))

    @pl.when(i == 0)
    def _init():
        acc_ref[0, 0] = 0.0
        acc_ref[0, 1] = 0.0

    acc_ref[0, 0] += bs
    acc_ref[0, 1] += bc

    @pl.when(i == NBLK - 1)
    def _fin():
        out_ref[...] = jnp.full((1, 1), acc_ref[0, 0] / acc_ref[0, 1],
                                dtype=jnp.float32)


def kernel(H, labels):
    lab_row = labels.reshape(ROWS, 1)
    lab_col = labels.reshape(1, ROWS)
    out = pl.pallas_call(
        _triplet_kernel,
        grid=(NBLK,),
        in_specs=[
            pl.BlockSpec((BLOCK, DIM), lambda i: (i, 0)),
            pl.BlockSpec((ROWS, DIM), lambda i: (0, 0)),
            pl.BlockSpec((BLOCK, 1), lambda i: (i, 0)),
            pl.BlockSpec((1, ROWS), lambda i: (0, 0)),
        ],
        out_specs=pl.BlockSpec((1, 1), lambda i: (0, 0)),
        out_shape=jax.ShapeDtypeStruct((1, 1), jnp.float32),
        scratch_shapes=[pltpu.SMEM((1, 2), jnp.float32)],
    )(H, H, lab_row, lab_col)
    return out[0, 0]
